# baseline (speedup 1.0000x reference)
import functools

import numpy as np
import jax
import jax.numpy as jnp
from jax import lax
from jax.experimental import pallas as pl
from jax.experimental.pallas import tpu as pltpu
from jax.experimental.pallas import tpu_sc as plsc

_PERIOD = np.float32(365.25)
_TWO_PI = np.float32(2.0 * np.pi)
_N = 64
_B = 16384

_NC = 2
_NS = 16
_NW = _NC * _NS
_BPW = _B // _NW

_BT = 1024
_GRID = _B // _BT


def _sc_gather(id_flat, a_table, b_table):
    mesh = plsc.VectorSubcoreMesh(core_axis_name="c", subcore_axis_name="s")

    @functools.partial(
        pl.kernel,
        mesh=mesh,
        out_type=[
            jax.ShapeDtypeStruct((_B, _N), jnp.float32),
            jax.ShapeDtypeStruct((_B, _N), jnp.float32),
        ],
        scratch_types=[
            pltpu.VMEM((_BPW,), jnp.int32),
            pltpu.SemaphoreType.DMA,
            pltpu.SemaphoreType.DMA,
        ],
    )
    def k(id_hbm, a_hbm, b_hbm, a_out, b_out, idx_v, sem_a, sem_b):
        wid = lax.axis_index("s") * _NC + lax.axis_index("c")
        base = wid * _BPW
        pltpu.sync_copy(id_hbm.at[pl.ds(base, _BPW)], idx_v)

        def body(j, carry):
            vec = idx_v[pl.ds(j * 16, 16)]
            for l in range(16):
                r = vec[l]
                i = j * 16 + l
                pltpu.async_copy(a_hbm.at[pl.ds(r, 1), :],
                                 a_out.at[pl.ds(base + i, 1), :], sem_a)
                pltpu.async_copy(b_hbm.at[pl.ds(r, 1), :],
                                 b_out.at[pl.ds(base + i, 1), :], sem_b)
            return carry

        lax.fori_loop(0, _BPW // 16, body, 0)
        pltpu.make_async_copy(a_hbm.at[pl.ds(0, _BPW)],
                              a_out.at[pl.ds(base, _BPW)], sem_a).wait()
        pltpu.make_async_copy(b_hbm.at[pl.ds(0, _BPW)],
                              b_out.at[pl.ds(base, _BPW)], sem_b).wait()

    return k(id_flat, a_table, b_table)


def _tc_body(t_ref, a_ref, b_ref, atab_ref, btab_ref, o_ref):
    n = (lax.broadcasted_iota(jnp.int32, (_BT, _N), 1) + 1).astype(jnp.float32)
    x = _TWO_PI * (_PERIOD * t_ref[...]) / n
    a0 = atab_ref[0:1, :]
    b0 = btab_ref[0:1, :]
    cos_part = jnp.cos(x) * (a_ref[...] + a0)
    sin_part = jnp.sin(x) * (b_ref[...] + b0)
    o_ref[...] = (jnp.sum(cos_part, axis=1, keepdims=True)
                  + jnp.sum(sin_part, axis=1, keepdims=True))


def _tc_fourier(t, a_rows, b_rows, a_table, b_table):
    return pl.pallas_call(
        _tc_body,
        grid=(_GRID,),
        in_specs=[
            pl.BlockSpec((_BT, 1), lambda i: (i, 0)),
            pl.BlockSpec((_BT, _N), lambda i: (i, 0)),
            pl.BlockSpec((_BT, _N), lambda i: (i, 0)),
            pl.BlockSpec((8, _N), lambda i: (0, 0)),
            pl.BlockSpec((8, _N), lambda i: (0, 0)),
        ],
        out_specs=pl.BlockSpec((_BT, 1), lambda i: (i, 0)),
        out_shape=jax.ShapeDtypeStruct((_B, 1), jnp.float32),
    )(t, a_rows, b_rows, a_table, b_table)


def kernel(id, t, a_table, b_table):
    id_flat = id.astype(jnp.int32).reshape(_B)
    a_rows, b_rows = _sc_gather(id_flat, a_table, b_table)
    return _tc_fourier(t, a_rows, b_rows, a_table, b_table)


# speedup vs baseline: 2.7984x; 2.7984x over previous
import functools

import numpy as np
import jax
import jax.numpy as jnp
from jax import lax
from jax.experimental import pallas as pl
from jax.experimental.pallas import tpu as pltpu
from jax.experimental.pallas import tpu_sc as plsc

_PERIOD = np.float32(365.25)
_TWO_PI = np.float32(2.0 * np.pi)
_N = 64
_B = 16384
_V = 100000

_NC = 2
_NS = 16
_NW = _NC * _NS
_BPW = _B // _NW
_CHUNK = 128
_NCHUNK = _BPW // _CHUNK

_BV = 1000
_BT = 1024
_GRID = _B // _BT


def _tc_fuse_body(a_ref, b_ref, o_ref):
    o_ref[...] = jnp.concatenate([a_ref[...], b_ref[...]], axis=1)


def _tc_fuse_tables(a_table, b_table):
    return pl.pallas_call(
        _tc_fuse_body,
        grid=(_V // _BV,),
        in_specs=[
            pl.BlockSpec((_BV, _N), lambda i: (i, 0)),
            pl.BlockSpec((_BV, _N), lambda i: (i, 0)),
        ],
        out_specs=pl.BlockSpec((_BV, 2 * _N), lambda i: (i, 0)),
        out_shape=jax.ShapeDtypeStruct((_V, 2 * _N), jnp.float32),
    )(a_table, b_table)


def _sc_gather(id_flat, ab_lin):
    mesh = plsc.VectorSubcoreMesh(core_axis_name="c", subcore_axis_name="s")

    @functools.partial(
        pl.kernel,
        mesh=mesh,
        compiler_params=pltpu.CompilerParams(use_tc_tiling_on_sc=False),
        out_type=jax.ShapeDtypeStruct((_B, 2 * _N), jnp.float32),
        scratch_types=[
            pltpu.VMEM((_BPW,), jnp.int32),
            pltpu.VMEM((_NCHUNK, _CHUNK, 2 * _N), jnp.float32),
            pltpu.SemaphoreType.DMA,
        ],
    )
    def k(id_hbm, ab_hbm, ab_out, idx_v, rows, sem):
        wid = lax.axis_index("s") * _NC + lax.axis_index("c")
        base = wid * _BPW
        pltpu.sync_copy(id_hbm.at[pl.ds(base, _BPW)], idx_v)
        copies = []
        for j in range(_NCHUNK):
            copies.append(pltpu.async_copy(
                ab_hbm.at[idx_v.at[pl.ds(j * _CHUNK, _CHUNK)]],
                rows.at[j], sem))
        for c in copies:
            c.wait()
        pltpu.sync_copy(rows.at[0], ab_out.at[pl.ds(base, _CHUNK)])
        pltpu.sync_copy(rows.at[1], ab_out.at[pl.ds(base + _CHUNK, _CHUNK)])
        pltpu.sync_copy(rows.at[2], ab_out.at[pl.ds(base + 2 * _CHUNK, _CHUNK)])
        pltpu.sync_copy(rows.at[3], ab_out.at[pl.ds(base + 3 * _CHUNK, _CHUNK)])

    return k(id_flat, ab_lin)


def _tc_body(t_ref, ab_ref, atab_ref, btab_ref, o_ref):
    j = lax.broadcasted_iota(jnp.int32, (_BT, 2 * _N), 1)
    n = ((j % _N) + 1).astype(jnp.float32)
    x = _TWO_PI * (_PERIOD * t_ref[...]) / n
    trig = jnp.where(j < _N, jnp.cos(x), jnp.sin(x))
    r0 = jnp.concatenate([atab_ref[0:1, :], btab_ref[0:1, :]], axis=1)
    o_ref[...] = jnp.sum(trig * (ab_ref[...] + r0), axis=1, keepdims=True)


def _tc_fourier(t, ab_rows, a_table, b_table):
    return pl.pallas_call(
        _tc_body,
        grid=(_GRID,),
        in_specs=[
            pl.BlockSpec((_BT, 1), lambda i: (i, 0)),
            pl.BlockSpec((_BT, 2 * _N), lambda i: (i, 0)),
            pl.BlockSpec((8, _N), lambda i: (0, 0)),
            pl.BlockSpec((8, _N), lambda i: (0, 0)),
        ],
        out_specs=pl.BlockSpec((_BT, 1), lambda i: (i, 0)),
        out_shape=jax.ShapeDtypeStruct((_B, 1), jnp.float32),
    )(t, ab_rows, a_table, b_table)


def kernel(id, t, a_table, b_table):
    id_flat = id.astype(jnp.int32).reshape(_B)
    ab_lin = _tc_fuse_tables(a_table, b_table)
    ab_rows = _sc_gather(id_flat, ab_lin)
    return _tc_fourier(t, ab_rows, a_table, b_table)


# speedup vs baseline: 4.7722x; 1.7053x over previous
import functools

import numpy as np
import jax
import jax.numpy as jnp
from jax import lax
from jax.experimental import pallas as pl
from jax.experimental.pallas import tpu as pltpu
from jax.experimental.pallas import tpu_sc as plsc

_PERIOD = np.float32(365.25)
_TWO_PI = np.float32(2.0 * np.pi)
_N = 64
_B = 16384
_V = 100000

_NC = 2
_NS = 16
_NW = _NC * _NS
_BPW = _B // _NW
_CHUNK = 128
_NCHUNK = _BPW // _CHUNK

_BV = 2048
_BT = 1024
_GRID = _B // _BT


def _tc_fuse_body(at_ref, bt_ref, o_ref):
    o_ref[...] = jnp.concatenate(
        [at_ref[...].T, bt_ref[...].T], axis=1)


def _tc_fuse_tables(a_t, b_t):
    return pl.pallas_call(
        _tc_fuse_body,
        grid=(pl.cdiv(_V, _BV),),
        in_specs=[
            pl.BlockSpec((_N, _BV), lambda i: (0, i)),
            pl.BlockSpec((_N, _BV), lambda i: (0, i)),
        ],
        out_specs=pl.BlockSpec((_BV, 2 * _N), lambda i: (i, 0)),
        out_shape=jax.ShapeDtypeStruct((_V, 2 * _N), jnp.float32),
    )(a_t, b_t)


def _sc_gather(id_flat, ab_lin):
    mesh = plsc.VectorSubcoreMesh(core_axis_name="c", subcore_axis_name="s")

    @functools.partial(
        pl.kernel,
        mesh=mesh,
        compiler_params=pltpu.CompilerParams(use_tc_tiling_on_sc=False),
        out_type=jax.ShapeDtypeStruct((_B, 2 * _N), jnp.float32),
        scratch_types=[
            pltpu.VMEM((_BPW,), jnp.int32),
            pltpu.VMEM((_NCHUNK, _CHUNK, 2 * _N), jnp.float32),
            pltpu.SemaphoreType.DMA,
        ],
    )
    def k(id_hbm, ab_hbm, ab_out, idx_v, rows, sem):
        wid = lax.axis_index("s") * _NC + lax.axis_index("c")
        base = wid * _BPW
        pltpu.sync_copy(id_hbm.at[pl.ds(base, _BPW)], idx_v)
        copies = []
        for j in range(_NCHUNK):
            copies.append(pltpu.async_copy(
                ab_hbm.at[idx_v.at[pl.ds(j * _CHUNK, _CHUNK)]],
                rows.at[j], sem))
        for c in copies:
            c.wait()
        pltpu.sync_copy(rows.at[0], ab_out.at[pl.ds(base, _CHUNK)])
        pltpu.sync_copy(rows.at[1], ab_out.at[pl.ds(base + _CHUNK, _CHUNK)])
        pltpu.sync_copy(rows.at[2], ab_out.at[pl.ds(base + 2 * _CHUNK, _CHUNK)])
        pltpu.sync_copy(rows.at[3], ab_out.at[pl.ds(base + 3 * _CHUNK, _CHUNK)])

    return k(id_flat, ab_lin)


def _tc_body(t_ref, ab_ref, abtab_ref, o_ref):
    j = lax.broadcasted_iota(jnp.int32, (_BT, 2 * _N), 1)
    n = ((j % _N) + 1).astype(jnp.float32)
    x = _TWO_PI * (_PERIOD * t_ref[...]) / n
    trig = jnp.where(j < _N, jnp.cos(x), jnp.sin(x))
    r0 = abtab_ref[0:1, :]
    o_ref[...] = jnp.sum(trig * (ab_ref[...] + r0), axis=1, keepdims=True)


def _tc_fourier(t, ab_rows, ab_lin):
    return pl.pallas_call(
        _tc_body,
        grid=(_GRID,),
        in_specs=[
            pl.BlockSpec((_BT, 1), lambda i: (i, 0)),
            pl.BlockSpec((_BT, 2 * _N), lambda i: (i, 0)),
            pl.BlockSpec((8, 2 * _N), lambda i: (0, 0)),
        ],
        out_specs=pl.BlockSpec((_BT, 1), lambda i: (i, 0)),
        out_shape=jax.ShapeDtypeStruct((_B, 1), jnp.float32),
    )(t, ab_rows, ab_lin)


def kernel(id, t, a_table, b_table):
    id_flat = id.astype(jnp.int32).reshape(_B)
    ab_lin = _tc_fuse_tables(a_table.T, b_table.T)
    ab_rows = _sc_gather(id_flat, ab_lin)
    return _tc_fourier(t, ab_rows, ab_lin)


# speedup vs baseline: 4.9438x; 1.0360x over previous
import functools

import numpy as np
import jax
import jax.numpy as jnp
from jax import lax
from jax.experimental import pallas as pl
from jax.experimental.pallas import tpu as pltpu
from jax.experimental.pallas import tpu_sc as plsc

_PERIOD = np.float32(365.25)
_TWO_PI = np.float32(2.0 * np.pi)
_N = 64
_B = 16384
_V = 100000

_NC = 2
_NS = 16
_NW = _NC * _NS
_BPW = _B // _NW
_CHUNK = 128
_NCHUNK = _BPW // _CHUNK

_FGRID = 32
_BV = 3200
_BTF = _B // _FGRID

_BT = 1024
_GRID = _B // _BT


def _tc_fuse_body(at_ref, bt_ref, at0_ref, bt0_ref, t_ref, ab_ref, trig_ref):
    a = at_ref[...].T
    b = bt_ref[...].T
    a0 = at0_ref[:, 0:1].T
    b0 = bt0_ref[:, 0:1].T
    ab_ref[...] = jnp.concatenate([a + a0, b + b0], axis=1)

    j = lax.broadcasted_iota(jnp.int32, (_BTF, 2 * _N), 1)
    n = ((j % _N) + 1).astype(jnp.float32)
    x = _TWO_PI * (_PERIOD * t_ref[...]) / n
    trig_ref[...] = jnp.where(j < _N, jnp.cos(x), jnp.sin(x))


def _tc_fuse_tables(a_t, b_t, t):
    return pl.pallas_call(
        _tc_fuse_body,
        grid=(_FGRID,),
        in_specs=[
            pl.BlockSpec((_N, _BV), lambda i: (0, i)),
            pl.BlockSpec((_N, _BV), lambda i: (0, i)),
            pl.BlockSpec((_N, 128), lambda i: (0, 0)),
            pl.BlockSpec((_N, 128), lambda i: (0, 0)),
            pl.BlockSpec((_BTF, 1), lambda i: (i, 0)),
        ],
        out_specs=[
            pl.BlockSpec((_BV, 2 * _N), lambda i: (i, 0)),
            pl.BlockSpec((_BTF, 2 * _N), lambda i: (i, 0)),
        ],
        out_shape=[
            jax.ShapeDtypeStruct((_V, 2 * _N), jnp.float32),
            jax.ShapeDtypeStruct((_B, 2 * _N), jnp.float32),
        ],
    )(a_t, b_t, a_t, b_t, t)


def _sc_gather(id_flat, ab_lin):
    mesh = plsc.VectorSubcoreMesh(core_axis_name="c", subcore_axis_name="s")

    @functools.partial(
        pl.kernel,
        mesh=mesh,
        compiler_params=pltpu.CompilerParams(use_tc_tiling_on_sc=False),
        out_type=jax.ShapeDtypeStruct((_B, 2 * _N), jnp.float32),
        scratch_types=[
            pltpu.VMEM((_BPW,), jnp.int32),
            pltpu.VMEM((_NCHUNK, _CHUNK, 2 * _N), jnp.float32),
            pltpu.SemaphoreType.DMA,
        ],
    )
    def k(id_hbm, ab_hbm, ab_out, idx_v, rows, sem):
        wid = lax.axis_index("s") * _NC + lax.axis_index("c")
        base = wid * _BPW
        pltpu.sync_copy(id_hbm.at[pl.ds(base, _BPW)], idx_v)
        copies = []
        for j in range(_NCHUNK):
            copies.append(pltpu.async_copy(
                ab_hbm.at[idx_v.at[pl.ds(j * _CHUNK, _CHUNK)]],
                rows.at[j], sem))
        for c in copies:
            c.wait()
        pltpu.sync_copy(rows.at[0], ab_out.at[pl.ds(base, _CHUNK)])
        pltpu.sync_copy(rows.at[1], ab_out.at[pl.ds(base + _CHUNK, _CHUNK)])
        pltpu.sync_copy(rows.at[2], ab_out.at[pl.ds(base + 2 * _CHUNK, _CHUNK)])
        pltpu.sync_copy(rows.at[3], ab_out.at[pl.ds(base + 3 * _CHUNK, _CHUNK)])

    return k(id_flat, ab_lin)


def _tc_reduce_body(trig_ref, ab_ref, o_ref):
    o_ref[...] = jnp.sum(trig_ref[...] * ab_ref[...], axis=1, keepdims=True)


def _tc_reduce(trig, ab_rows):
    return pl.pallas_call(
        _tc_reduce_body,
        grid=(_GRID,),
        in_specs=[
            pl.BlockSpec((_BT, 2 * _N), lambda i: (i, 0)),
            pl.BlockSpec((_BT, 2 * _N), lambda i: (i, 0)),
        ],
        out_specs=pl.BlockSpec((_BT, 1), lambda i: (i, 0)),
        out_shape=jax.ShapeDtypeStruct((_B, 1), jnp.float32),
    )(trig, ab_rows)


def kernel(id, t, a_table, b_table):
    id_flat = id.astype(jnp.int32).reshape(_B)
    ab_lin, trig = _tc_fuse_tables(a_table.T, b_table.T, t)
    ab_rows = _sc_gather(id_flat, ab_lin)
    return _tc_reduce(trig, ab_rows)


# speedup vs baseline: 5.8388x; 1.1810x over previous
import functools

import numpy as np
import jax
import jax.numpy as jnp
from jax import lax
from jax.experimental import pallas as pl
from jax.experimental.pallas import tpu as pltpu
from jax.experimental.pallas import tpu_sc as plsc

_PERIOD = np.float32(365.25)
_TWO_PI = np.float32(2.0 * np.pi)
_N = 64
_B = 16384
_V = 100000

_NC = 2
_NS = 16
_NW = _NC * _NS
_BPW = _B // _NW
_CHUNK = 128
_NCHUNK = _BPW // _CHUNK

_FGRID = 32
_BV = 3200
_BTF = _B // _FGRID

_BT = 1024
_GRID = _B // _BT


_INV2PI = np.float32(1.0 / (2.0 * np.pi))
_RC1 = np.float32(6.25)
_RC2 = np.float32(2.0 * np.pi - 6.25)
_COS_P = [np.float32(c) for c in (
    1.0, -0.49999991059303284, 0.04166650027036667, -0.0013887850800529122,
    2.4770743038970977e-05, -2.708605109091877e-07, 1.726593534812082e-09)]
_SIN_P = [np.float32(c) for c in (
    1.0, -0.1666666567325592, 0.008333321660757065, -0.0001984056580113247,
    2.7536434572539292e-06, -2.473357341159499e-08, 1.3627225736723148e-10)]


def _poly(s, coeffs):
    acc = jnp.full_like(s, coeffs[-1])
    for c in coeffs[-2::-1]:
        acc = acc * s + c
    return acc


def _tc_fuse_body(at_ref, bt_ref, at0_ref, bt0_ref, t_ref, ab_ref, trig_ref):
    a = at_ref[...].T
    b = bt_ref[...].T
    a0 = at0_ref[:, 0:1].T
    b0 = bt0_ref[:, 0:1].T
    ab_ref[...] = jnp.concatenate([a + a0, b + b0], axis=1)

    j = lax.broadcasted_iota(jnp.int32, (_BTF, 2 * _N), 1)
    n = ((j % _N) + 1).astype(jnp.float32)
    x = _TWO_PI * (_PERIOD * t_ref[...]) / n
    k = jnp.floor(x * _INV2PI + 0.5)
    r = (x - k * _RC1) - k * _RC2
    s = r * r
    cosv = _poly(s, _COS_P)
    sinv = r * _poly(s, _SIN_P)
    trig_ref[...] = jnp.where(j < _N, cosv, sinv).astype(jnp.bfloat16)


def _tc_fuse_tables(a_t, b_t, t):
    return pl.pallas_call(
        _tc_fuse_body,
        grid=(_FGRID,),
        in_specs=[
            pl.BlockSpec((_N, _BV), lambda i: (0, i)),
            pl.BlockSpec((_N, _BV), lambda i: (0, i)),
            pl.BlockSpec((_N, 128), lambda i: (0, 0)),
            pl.BlockSpec((_N, 128), lambda i: (0, 0)),
            pl.BlockSpec((_BTF, 1), lambda i: (i, 0)),
        ],
        out_specs=[
            pl.BlockSpec((_BV, 2 * _N), lambda i: (i, 0)),
            pl.BlockSpec((_BTF, 2 * _N), lambda i: (i, 0)),
        ],
        out_shape=[
            jax.ShapeDtypeStruct((_V, 2 * _N), jnp.float32),
            jax.ShapeDtypeStruct((_B, 2 * _N), jnp.bfloat16),
        ],
    )(a_t, b_t, a_t, b_t, t)


def _sc_gather(id_flat, ab_lin):
    mesh = plsc.VectorSubcoreMesh(core_axis_name="c", subcore_axis_name="s")

    @functools.partial(
        pl.kernel,
        mesh=mesh,
        compiler_params=pltpu.CompilerParams(use_tc_tiling_on_sc=False),
        out_type=jax.ShapeDtypeStruct((_B, 2 * _N), jnp.float32),
        scratch_types=[
            pltpu.VMEM((_BPW,), jnp.int32),
            pltpu.VMEM((_NCHUNK, _CHUNK, 2 * _N), jnp.float32),
            pltpu.SemaphoreType.DMA,
        ],
    )
    def k(id_hbm, ab_hbm, ab_out, idx_v, rows, sem):
        wid = lax.axis_index("s") * _NC + lax.axis_index("c")
        base = wid * _BPW
        pltpu.sync_copy(id_hbm.at[pl.ds(base, _BPW)], idx_v)
        copies = []
        for j in range(_NCHUNK):
            copies.append(pltpu.async_copy(
                ab_hbm.at[idx_v.at[pl.ds(j * _CHUNK, _CHUNK)]],
                rows.at[j], sem))
        for c in copies:
            c.wait()
        pltpu.sync_copy(rows.at[0], ab_out.at[pl.ds(base, _CHUNK)])
        pltpu.sync_copy(rows.at[1], ab_out.at[pl.ds(base + _CHUNK, _CHUNK)])
        pltpu.sync_copy(rows.at[2], ab_out.at[pl.ds(base + 2 * _CHUNK, _CHUNK)])
        pltpu.sync_copy(rows.at[3], ab_out.at[pl.ds(base + 3 * _CHUNK, _CHUNK)])

    return k(id_flat, ab_lin)


def _tc_reduce_body(trig_ref, ab_ref, o_ref):
    trig = trig_ref[...].astype(jnp.float32)
    o_ref[...] = jnp.sum(trig * ab_ref[...], axis=1)


def _tc_reduce(trig, ab_rows):
    return pl.pallas_call(
        _tc_reduce_body,
        grid=(_GRID,),
        in_specs=[
            pl.BlockSpec((_BT, 2 * _N), lambda i: (i, 0)),
            pl.BlockSpec((_BT, 2 * _N), lambda i: (i, 0)),
        ],
        out_specs=pl.BlockSpec((_BT,), lambda i: (i,)),
        out_shape=jax.ShapeDtypeStruct((_B,), jnp.float32),
    )(trig, ab_rows)


def kernel(id, t, a_table, b_table):
    id_flat = id.astype(jnp.int32).reshape(_B)
    ab_lin, trig = _tc_fuse_tables(a_table.T, b_table.T, t)
    ab_rows = _sc_gather(id_flat, ab_lin)
    return _tc_reduce(trig, ab_rows).reshape(_B, 1)


# speedup vs baseline: 6.6435x; 1.1378x over previous
import functools

import numpy as np
import jax
import jax.numpy as jnp
from jax import lax
from jax.experimental import pallas as pl
from jax.experimental.pallas import tpu as pltpu
from jax.experimental.pallas import tpu_sc as plsc

_PERIOD = np.float32(365.25)
_TWO_PI = np.float32(2.0 * np.pi)
_N = 64
_B = 16384
_V = 100000
_VSPLIT = 51200

_NC = 2
_NS = 16
_NW = _NC * _NS
_BPW = _B // _NW
_CHUNK = 128
_NCHUNK = _BPW // _CHUNK

_FGRID = 16
_BVH = 3200
_BTF = _B // _FGRID

_BT = 2048
_GRID = _B // _BT

_INV2PI = np.float32(1.0 / (2.0 * np.pi))
_RC1 = np.float32(6.25)
_RC2 = np.float32(2.0 * np.pi - 6.25)
_COS_P = [np.float32(c) for c in (
    1.0, -0.49999991059303284, 0.04166650027036667, -0.0013887850800529122,
    2.4770743038970977e-05, -2.708605109091877e-07, 1.726593534812082e-09)]
_SIN_P = [np.float32(c) for c in (
    1.0, -0.1666666567325592, 0.008333321660757065, -0.0001984056580113247,
    2.7536434572539292e-06, -2.473357341159499e-08, 1.3627225736723148e-10)]


def _poly(s, coeffs):
    acc = jnp.full_like(s, coeffs[-1])
    for c in coeffs[-2::-1]:
        acc = acc * s + c
    return acc


def _bf16_bits(x):
    xi = lax.bitcast_convert_type(x, jnp.int32)
    lsb = lax.shift_right_logical(xi, 16) & 1
    return lax.shift_right_logical(xi + 0x7FFF + lsb, 16) & 0xFFFF


def _tc_fuse_body(at1_ref, bt1_ref, at2_ref, bt2_ref, at0_ref, bt0_ref,
                  tq_ref, pk_ref, tl_ref, th_ref):
    a0 = at0_ref[:, 0:1].T
    b0 = bt0_ref[:, 0:1].T
    a1 = at1_ref[...].T + a0
    b1 = bt1_ref[...].T + b0
    a2 = at2_ref[...].T + a0
    b2 = bt2_ref[...].T + b0
    w1 = lax.shift_left(_bf16_bits(b1), 16) | _bf16_bits(a1)
    w2 = lax.shift_left(_bf16_bits(b2), 16) | _bf16_bits(a2)
    pk_ref[...] = lax.bitcast_convert_type(
        jnp.concatenate([w1, w2], axis=1), jnp.float32)

    t = tq_ref[:, 0:1]
    par = tq_ref[:, 1:2]
    j = lax.broadcasted_iota(jnp.int32, (_BTF, 2 * _N), 1)
    n = ((j % _N) + 1).astype(jnp.float32)
    x = _TWO_PI * (_PERIOD * t) / n
    k = jnp.floor(x * _INV2PI + 0.5)
    r = (x - k * _RC1) - k * _RC2
    s = r * r
    cosv = _poly(s, _COS_P)
    sinv = r * _poly(s, _SIN_P)
    half = (j >= _N).astype(jnp.float32)
    pm = 1.0 - jnp.abs(half - par)
    tl_ref[...] = (cosv * pm).astype(jnp.bfloat16)
    th_ref[...] = (sinv * pm).astype(jnp.bfloat16)


def _tc_fuse_tables(a_t, b_t, tq):
    return pl.pallas_call(
        _tc_fuse_body,
        grid=(_FGRID,),
        in_specs=[
            pl.BlockSpec((_N, _BVH), lambda i: (0, i)),
            pl.BlockSpec((_N, _BVH), lambda i: (0, i)),
            pl.BlockSpec((_N, _BVH), lambda i: (0, i + _FGRID)),
            pl.BlockSpec((_N, _BVH), lambda i: (0, i + _FGRID)),
            pl.BlockSpec((_N, 128), lambda i: (0, 0)),
            pl.BlockSpec((_N, 128), lambda i: (0, 0)),
            pl.BlockSpec((_BTF, 2), lambda i: (i, 0)),
        ],
        out_specs=[
            pl.BlockSpec((_BVH, 2 * _N), lambda i: (i, 0)),
            pl.BlockSpec((_BTF, 2 * _N), lambda i: (i, 0)),
            pl.BlockSpec((_BTF, 2 * _N), lambda i: (i, 0)),
        ],
        out_shape=[
            jax.ShapeDtypeStruct((_VSPLIT, 2 * _N), jnp.float32),
            jax.ShapeDtypeStruct((_B, 2 * _N), jnp.bfloat16),
            jax.ShapeDtypeStruct((_B, 2 * _N), jnp.bfloat16),
        ],
    )(a_t, b_t, a_t, b_t, a_t, b_t, tq)


def _sc_gather(idm, ab_pack):
    mesh = plsc.VectorSubcoreMesh(core_axis_name="c", subcore_axis_name="s")

    @functools.partial(
        pl.kernel,
        mesh=mesh,
        compiler_params=pltpu.CompilerParams(use_tc_tiling_on_sc=False),
        out_type=jax.ShapeDtypeStruct((_B, 2 * _N), jnp.float32),
        scratch_types=[
            pltpu.VMEM((_BPW,), jnp.int32),
            pltpu.VMEM((_NCHUNK, _CHUNK, 2 * _N), jnp.float32),
            pltpu.SemaphoreType.DMA,
        ],
    )
    def k(id_hbm, ab_hbm, ab_out, idx_v, rows, sem):
        wid = lax.axis_index("s") * _NC + lax.axis_index("c")
        base = wid * _BPW
        pltpu.sync_copy(id_hbm.at[pl.ds(base, _BPW)], idx_v)
        copies = []
        for j in range(_NCHUNK):
            copies.append(pltpu.async_copy(
                ab_hbm.at[idx_v.at[pl.ds(j * _CHUNK, _CHUNK)]],
                rows.at[j], sem))
        for c in copies:
            c.wait()
        pltpu.sync_copy(rows.at[0], ab_out.at[pl.ds(base, _CHUNK)])
        pltpu.sync_copy(rows.at[1], ab_out.at[pl.ds(base + _CHUNK, _CHUNK)])
        pltpu.sync_copy(rows.at[2], ab_out.at[pl.ds(base + 2 * _CHUNK, _CHUNK)])
        pltpu.sync_copy(rows.at[3], ab_out.at[pl.ds(base + 3 * _CHUNK, _CHUNK)])

    return k(idm, ab_pack)


def _tc_reduce_body(tl_ref, th_ref, ab_ref, o_ref):
    w = lax.bitcast_convert_type(ab_ref[...], jnp.int32)
    lo = lax.bitcast_convert_type(lax.shift_left(w, 16), jnp.float32)
    hi = lax.bitcast_convert_type(w & np.int32(-65536), jnp.float32)
    tl = tl_ref[...].astype(jnp.float32)
    th = th_ref[...].astype(jnp.float32)
    o_ref[...] = jnp.sum(tl * lo + th * hi, axis=1)


def _tc_reduce(tl, th, ab_rows):
    return pl.pallas_call(
        _tc_reduce_body,
        grid=(_GRID,),
        in_specs=[
            pl.BlockSpec((_BT, 2 * _N), lambda i: (i, 0)),
            pl.BlockSpec((_BT, 2 * _N), lambda i: (i, 0)),
            pl.BlockSpec((_BT, 2 * _N), lambda i: (i, 0)),
        ],
        out_specs=pl.BlockSpec((_BT,), lambda i: (i,)),
        out_shape=jax.ShapeDtypeStruct((_B,), jnp.float32),
    )(tl, th, ab_rows)


def kernel(id, t, a_table, b_table):
    id_flat = id.astype(jnp.int32).reshape(_B)
    par = (id_flat >= _VSPLIT)
    idm = jnp.where(par, id_flat - _VSPLIT, id_flat)
    tq = jnp.concatenate(
        [t, par.astype(jnp.float32).reshape(_B, 1)], axis=1)
    ab_pack, tl, th = _tc_fuse_tables(a_table.T, b_table.T, tq)
    ab_rows = _sc_gather(idm, ab_pack)
    return _tc_reduce(tl, th, ab_rows).reshape(_B, 1)


# speedup vs baseline: 6.7143x; 1.0107x over previous
import functools

import numpy as np
import jax
import jax.numpy as jnp
from jax import lax
from jax.experimental import pallas as pl
from jax.experimental.pallas import tpu as pltpu
from jax.experimental.pallas import tpu_sc as plsc

_PERIOD = np.float32(365.25)
_TWO_PI = np.float32(2.0 * np.pi)
_N = 64
_B = 16384
_V = 100000
_VSPLIT = 51200

_NC = 2
_NS = 16
_NW = _NC * _NS
_BPW = _B // _NW
_CHUNK = 128
_NCHUNK = _BPW // _CHUNK

_FGRID = 8
_BVH = 6400
_BTF = _B // _FGRID

_BT = 4096
_GRID = _B // _BT

_INV2PI = np.float32(1.0 / (2.0 * np.pi))
_RC1 = np.float32(6.25)
_RC2 = np.float32(2.0 * np.pi - 6.25)
_COS_P = [np.float32(c) for c in (
    1.0, -0.49999991059303284, 0.04166650027036667, -0.0013887850800529122,
    2.4770743038970977e-05, -2.708605109091877e-07, 1.726593534812082e-09)]
_SIN_P = [np.float32(c) for c in (
    1.0, -0.1666666567325592, 0.008333321660757065, -0.0001984056580113247,
    2.7536434572539292e-06, -2.473357341159499e-08, 1.3627225736723148e-10)]


def _poly(s, coeffs):
    acc = jnp.full_like(s, coeffs[-1])
    for c in coeffs[-2::-1]:
        acc = acc * s + c
    return acc


def _bf16_bits(x):
    xi = lax.bitcast_convert_type(x, jnp.int32)
    lsb = lax.shift_right_logical(xi, 16) & 1
    return lax.shift_right_logical(xi + 0x7FFF + lsb, 16) & 0xFFFF


def _tc_fuse_body(at1_ref, bt1_ref, at2_ref, bt2_ref, at0_ref, bt0_ref,
                  tq_ref, pk_ref, tl_ref, th_ref):
    a0 = at0_ref[:, 0:1].T
    b0 = bt0_ref[:, 0:1].T
    a1 = at1_ref[...].T + a0
    b1 = bt1_ref[...].T + b0
    a2 = at2_ref[...].T + a0
    b2 = bt2_ref[...].T + b0
    w1 = lax.shift_left(_bf16_bits(b1), 16) | _bf16_bits(a1)
    w2 = lax.shift_left(_bf16_bits(b2), 16) | _bf16_bits(a2)
    pk_ref[...] = lax.bitcast_convert_type(
        jnp.concatenate([w1, w2], axis=1), jnp.float32)

    t = tq_ref[:, 0:1]
    par = tq_ref[:, 1:2]
    j = lax.broadcasted_iota(jnp.int32, (_BTF, 2 * _N), 1)
    n = ((j % _N) + 1).astype(jnp.float32)
    x = _TWO_PI * (_PERIOD * t) / n
    k = jnp.floor(x * _INV2PI + 0.5)
    r = (x - k * _RC1) - k * _RC2
    s = r * r
    cosv = _poly(s, _COS_P)
    sinv = r * _poly(s, _SIN_P)
    half = (j >= _N).astype(jnp.float32)
    pm = 1.0 - jnp.abs(half - par)
    tl_ref[...] = (cosv * pm).astype(jnp.bfloat16)
    th_ref[...] = (sinv * pm).astype(jnp.bfloat16)


def _tc_fuse_tables(a_t, b_t, tq):
    return pl.pallas_call(
        _tc_fuse_body,
        grid=(_FGRID,),
        in_specs=[
            pl.BlockSpec((_N, _BVH), lambda i: (0, i)),
            pl.BlockSpec((_N, _BVH), lambda i: (0, i)),
            pl.BlockSpec((_N, _BVH), lambda i: (0, i + _FGRID)),
            pl.BlockSpec((_N, _BVH), lambda i: (0, i + _FGRID)),
            pl.BlockSpec((_N, 128), lambda i: (0, 0)),
            pl.BlockSpec((_N, 128), lambda i: (0, 0)),
            pl.BlockSpec((_BTF, 2), lambda i: (i, 0)),
        ],
        out_specs=[
            pl.BlockSpec((_BVH, 2 * _N), lambda i: (i, 0)),
            pl.BlockSpec((_BTF, 2 * _N), lambda i: (i, 0)),
            pl.BlockSpec((_BTF, 2 * _N), lambda i: (i, 0)),
        ],
        out_shape=[
            jax.ShapeDtypeStruct((_VSPLIT, 2 * _N), jnp.float32),
            jax.ShapeDtypeStruct((_B, 2 * _N), jnp.bfloat16),
            jax.ShapeDtypeStruct((_B, 2 * _N), jnp.bfloat16),
        ],
    )(a_t, b_t, a_t, b_t, a_t, b_t, tq)


def _sc_gather(idm, ab_pack):
    mesh = plsc.VectorSubcoreMesh(core_axis_name="c", subcore_axis_name="s")

    @functools.partial(
        pl.kernel,
        mesh=mesh,
        compiler_params=pltpu.CompilerParams(use_tc_tiling_on_sc=False),
        out_type=jax.ShapeDtypeStruct((_B, 2 * _N), jnp.float32),
        scratch_types=[
            pltpu.VMEM((_BPW,), jnp.int32),
            pltpu.VMEM((_NCHUNK, _CHUNK, 2 * _N), jnp.float32),
            pltpu.SemaphoreType.DMA,
        ],
    )
    def k(id_hbm, ab_hbm, ab_out, idx_v, rows, sem):
        wid = lax.axis_index("s") * _NC + lax.axis_index("c")
        base = wid * _BPW
        pltpu.sync_copy(id_hbm.at[pl.ds(base, _BPW)], idx_v)
        copies = []
        for j in range(_NCHUNK):
            copies.append(pltpu.async_copy(
                ab_hbm.at[idx_v.at[pl.ds(j * _CHUNK, _CHUNK)]],
                rows.at[j], sem))
        for c in copies:
            c.wait()
        pltpu.sync_copy(rows.at[0], ab_out.at[pl.ds(base, _CHUNK)])
        pltpu.sync_copy(rows.at[1], ab_out.at[pl.ds(base + _CHUNK, _CHUNK)])
        pltpu.sync_copy(rows.at[2], ab_out.at[pl.ds(base + 2 * _CHUNK, _CHUNK)])
        pltpu.sync_copy(rows.at[3], ab_out.at[pl.ds(base + 3 * _CHUNK, _CHUNK)])

    return k(idm, ab_pack)


def _tc_reduce_body(tl_ref, th_ref, ab_ref, o_ref):
    w = lax.bitcast_convert_type(ab_ref[...], jnp.int32)
    lo = lax.bitcast_convert_type(lax.shift_left(w, 16), jnp.float32)
    hi = lax.bitcast_convert_type(w & np.int32(-65536), jnp.float32)
    tl = tl_ref[...].astype(jnp.float32)
    th = th_ref[...].astype(jnp.float32)
    o_ref[...] = jnp.sum(tl * lo + th * hi, axis=1)


def _tc_reduce(tl, th, ab_rows):
    return pl.pallas_call(
        _tc_reduce_body,
        grid=(_GRID,),
        in_specs=[
            pl.BlockSpec((_BT, 2 * _N), lambda i: (i, 0)),
            pl.BlockSpec((_BT, 2 * _N), lambda i: (i, 0)),
            pl.BlockSpec((_BT, 2 * _N), lambda i: (i, 0)),
        ],
        out_specs=pl.BlockSpec((_BT,), lambda i: (i,)),
        out_shape=jax.ShapeDtypeStruct((_B,), jnp.float32),
    )(tl, th, ab_rows)


def kernel(id, t, a_table, b_table):
    id_flat = id.astype(jnp.int32).reshape(_B)
    par = (id_flat >= _VSPLIT)
    idm = jnp.where(par, id_flat - _VSPLIT, id_flat)
    tq = jnp.concatenate(
        [t, par.astype(jnp.float32).reshape(_B, 1)], axis=1)
    ab_pack, tl, th = _tc_fuse_tables(a_table.T, b_table.T, tq)
    ab_rows = _sc_gather(idm, ab_pack)
    return _tc_reduce(tl, th, ab_rows).reshape(_B, 1)
